# baseline (speedup 1.0000x reference)
import functools

import jax
import jax.numpy as jnp
from jax import lax
from jax.experimental import pallas as pl
from jax.experimental.pallas import tpu as pltpu
from jax.experimental.pallas import tpu_sc as plsc

_L = 16


def _newton_rsqrt(s):
    i = lax.bitcast_convert_type(s, jnp.int32)
    i = jnp.int32(0x5F3759DF) - lax.shift_right_logical(i, jnp.int32(1))
    y = lax.bitcast_convert_type(i, jnp.float32)
    half = jnp.float32(0.5)
    three_half = jnp.float32(1.5)
    for _ in range(3):
        y = y * (three_half - half * s * y * y)
    return y


def kernel(all_rays, indices, P_in, V_in):
    n_tab = all_rays.shape[0]
    B = indices.shape[0]
    n_pre = P_in.shape[0]
    info = plsc.get_sparse_core_info()
    NC, NS = info.num_cores, info.num_subcores
    NW = NC * NS
    mesh_kw = dict(core_axis_name="c", subcore_axis_name="s")

    KC = 8000
    n_ck = n_tab // KC

    @functools.partial(
        pl.kernel,
        out_type=jax.ShapeDtypeStruct((n_tab * 8,), jnp.float32),
        mesh=plsc.VectorSubcoreMesh(**mesh_kw),
        scratch_types=[
            pltpu.VMEM((KC * 6,), jnp.float32),
            pltpu.VMEM((KC * 8,), jnp.float32),
        ],
        compiler_params=pltpu.CompilerParams(
            needs_layout_passes=False, use_tc_tiling_on_sc=False),
    )
    def pack(rays, t8, in_v, out_v):
        wid = lax.axis_index("s") * NC + lax.axis_index("c")
        lane = lax.iota(jnp.int32, _L)
        zvec = jnp.zeros((_L,), jnp.float32)

        def zinit(k, carry):
            out_v[pl.ds(k * _L, _L)] = zvec
            return carry

        lax.fori_loop(0, (KC * 8) // _L, zinit, 0)

        def chunk_body(ck, carry):
            c = wid + ck * NW

            @pl.when(c < n_ck)
            def _():
                start = c * KC
                pltpu.sync_copy(rays.at[pl.ds(start * 6, KC * 6)], in_v)

                def grp(i, carry2):
                    ray = lane + i * _L
                    src = ray * 6

                    def comp(j):
                        return plsc.load_gather(in_v, [src + j])

                    r0 = comp(0)
                    r1 = comp(1)
                    r3 = comp(3)
                    r4 = comp(4)
                    r5 = comp(5)
                    s = r3 * r3 + r4 * r4 + r5 * r5
                    y = _newton_rsqrt(s)
                    norm = s * y
                    inv = jnp.float32(1.0) / jnp.maximum(
                        norm, jnp.float32(1e-12))
                    d = ray * 8
                    plsc.store_scatter(out_v, [d + 1], jnp.float32(1000.0) * r0)
                    plsc.store_scatter(out_v, [d + 2], jnp.float32(1000.0) * r1)
                    plsc.store_scatter(out_v, [d + 3], -r5 * inv)
                    plsc.store_scatter(out_v, [d + 4], r3 * inv)
                    plsc.store_scatter(out_v, [d + 5], r4 * inv)
                    return carry2

                lax.fori_loop(0, KC // _L, grp, 0)
                pltpu.sync_copy(out_v, t8.at[pl.ds(start * 8, KC * 8)])

            return carry

        lax.fori_loop(0, (n_ck + NW - 1) // NW, chunk_body, 0)

    t8 = pack(all_rays.reshape(n_tab * 6)).reshape(n_tab, 8)

    R = B // NW
    C = 4096
    n_chunks = R // C
    n_out = B + n_pre
    out_sds = jax.ShapeDtypeStruct((n_out * 3,), jnp.float32)

    @functools.partial(
        pl.kernel,
        out_type=(out_sds, out_sds),
        mesh=plsc.VectorSubcoreMesh(**mesh_kw),
        scratch_types=[
            pltpu.VMEM((C,), jnp.int32),
            pltpu.VMEM((C, 8), jnp.float32),
            pltpu.VMEM((C * 3,), jnp.float32),
            pltpu.VMEM((C * 3,), jnp.float32),
            pltpu.SemaphoreType.DMA,
        ],
        compiler_params=pltpu.CompilerParams(
            needs_layout_passes=False, use_tc_tiling_on_sc=False),
    )
    def sample(table, idx, p_in, v_in, p_out, v_out,
               idx_v, rows_v, ps_v, vs_v, sem):
        wid = lax.axis_index("s") * NC + lax.axis_index("c")
        lane = lax.iota(jnp.int32, _L)
        zvec = jnp.zeros((_L,), jnp.float32)

        def zinit(k, carry):
            ps_v[pl.ds(k * _L, _L)] = zvec
            return carry

        lax.fori_loop(0, (C * 3) // _L, zinit, 0)

        @pl.when(wid == 0)
        def _():
            stage = vs_v.at[pl.ds(0, n_pre * 3)]
            pltpu.sync_copy(p_in, stage)
            pltpu.sync_copy(stage, p_out.at[pl.ds(0, n_pre * 3)])
            pltpu.sync_copy(v_in, stage)
            pltpu.sync_copy(stage, v_out.at[pl.ds(0, n_pre * 3)])

        def chunk_body(g, carry):
            base = wid * R + g * C
            pltpu.sync_copy(idx.at[pl.ds(base, C)], idx_v)
            pltpu.async_copy(table.at[idx_v], rows_v, sem).wait()

            def grp(i, carry2):
                s = lane + i * _L
                d = s * 3

                def col(j):
                    return plsc.load_gather(
                        rows_v, [s, jnp.full((_L,), j, jnp.int32)])

                plsc.store_scatter(ps_v, [d + 1], col(1))
                plsc.store_scatter(ps_v, [d + 2], col(2))
                plsc.store_scatter(vs_v, [d], col(3))
                plsc.store_scatter(vs_v, [d + 1], col(4))
                plsc.store_scatter(vs_v, [d + 2], col(5))
                return carry2

            lax.fori_loop(0, C // _L, grp, 0)
            dst = pl.ds((n_pre + base) * 3, C * 3)
            pltpu.sync_copy(ps_v, p_out.at[dst])
            pltpu.sync_copy(vs_v, v_out.at[dst])
            return carry

        lax.fori_loop(0, n_chunks, chunk_body, 0)

    p_flat, v_flat = sample(
        t8, indices.astype(jnp.int32),
        P_in.reshape(n_pre * 3), V_in.reshape(n_pre * 3))
    return (p_flat.reshape(n_out, 3), v_flat.reshape(n_out, 3))


# speedup vs baseline: 2.5763x; 2.5763x over previous
import functools

import jax
import jax.numpy as jnp
from jax import lax
from jax.experimental import pallas as pl
from jax.experimental.pallas import tpu as pltpu
from jax.experimental.pallas import tpu_sc as plsc

_L = 16


def _newton_rsqrt(s):
    i = lax.bitcast_convert_type(s, jnp.int32)
    i = jnp.int32(0x5F3759DF) - lax.shift_right_logical(i, jnp.int32(1))
    y = lax.bitcast_convert_type(i, jnp.float32)
    half = jnp.float32(0.5)
    three_half = jnp.float32(1.5)
    for _ in range(3):
        y = y * (three_half - half * s * y * y)
    return y


def kernel(all_rays, indices, P_in, V_in):
    n_tab = all_rays.shape[0]
    B = indices.shape[0]
    n_pre = P_in.shape[0]
    info = plsc.get_sparse_core_info()
    NC, NS = info.num_cores, info.num_subcores
    NW = NC * NS
    mesh_kw = dict(core_axis_name="c", subcore_axis_name="s")

    KC = 8000
    n_ck = n_tab // KC
    comp_of = [0, 1, 3, 4, 5]

    @functools.partial(
        pl.kernel,
        out_type=jax.ShapeDtypeStruct((n_tab * 8,), jnp.float32),
        mesh=plsc.VectorSubcoreMesh(**mesh_kw),
        scratch_types=[
            pltpu.VMEM((5 * KC,), jnp.float32),
            pltpu.VMEM((KC * 8,), jnp.float32),
        ],
        compiler_params=pltpu.CompilerParams(
            needs_layout_passes=False, use_tc_tiling_on_sc=False),
    )
    def pack(rays_t, t8, in_v, out_v):
        wid = lax.axis_index("s") * NC + lax.axis_index("c")
        lane = lax.iota(jnp.int32, _L)
        zvec = jnp.zeros((_L,), jnp.float32)

        def zinit(k, carry):
            out_v[pl.ds(k * _L, _L)] = zvec
            return carry

        lax.fori_loop(0, (KC * 8) // _L, zinit, 0)

        def chunk_body(ck, carry):
            c = wid + ck * NW

            @pl.when(c < n_ck)
            def _():
                start = c * KC
                for jj, j in enumerate(comp_of):
                    pltpu.sync_copy(
                        rays_t.at[pl.ds(j * n_tab + start, KC)],
                        in_v.at[pl.ds(jj * KC, KC)])

                def grp(i, carry2):
                    off = i * _L

                    def comp(jj):
                        return in_v[pl.ds(jj * KC + off, _L)]

                    r0 = comp(0)
                    r1 = comp(1)
                    r3 = comp(2)
                    r4 = comp(3)
                    r5 = comp(4)
                    s = r3 * r3 + r4 * r4 + r5 * r5
                    y = _newton_rsqrt(s)
                    norm = s * y
                    inv = jnp.float32(1.0) / jnp.maximum(
                        norm, jnp.float32(1e-12))
                    d = (lane + off) * 8
                    plsc.store_scatter(out_v, [d + 1], jnp.float32(1000.0) * r0)
                    plsc.store_scatter(out_v, [d + 2], jnp.float32(1000.0) * r1)
                    plsc.store_scatter(out_v, [d + 3], -r5 * inv)
                    plsc.store_scatter(out_v, [d + 4], r3 * inv)
                    plsc.store_scatter(out_v, [d + 5], r4 * inv)
                    return carry2

                lax.fori_loop(0, KC // _L, grp, 0)
                pltpu.sync_copy(out_v, t8.at[pl.ds(start * 8, KC * 8)])

            return carry

        lax.fori_loop(0, (n_ck + NW - 1) // NW, chunk_body, 0)

    t8 = pack(all_rays.T.reshape(n_tab * 6)).reshape(n_tab, 8)

    R = B // NW
    C = 4096
    n_chunks = R // C
    n_out = B + n_pre
    out_sds = jax.ShapeDtypeStruct((3, n_out), jnp.float32)

    @functools.partial(
        pl.kernel,
        out_type=(out_sds, out_sds),
        mesh=plsc.VectorSubcoreMesh(**mesh_kw),
        scratch_types=[
            pltpu.VMEM((C,), jnp.int32),
            pltpu.VMEM((C, 8), jnp.float32),
            pltpu.VMEM((5, C), jnp.float32),
            pltpu.VMEM((C,), jnp.float32),
            pltpu.SemaphoreType.DMA,
        ],
        compiler_params=pltpu.CompilerParams(
            needs_layout_passes=False, use_tc_tiling_on_sc=False),
    )
    def sample(table, idx, p_in_t, v_in_t, p_out, v_out,
               idx_v, rows_v, soa_v, zero_v, sem):
        wid = lax.axis_index("s") * NC + lax.axis_index("c")
        zvec = jnp.zeros((_L,), jnp.float32)

        def zinit(i, carry):
            zero_v[pl.ds(i * _L, _L)] = zvec
            return carry

        lax.fori_loop(0, C // _L, zinit, 0)

        @pl.when(wid == 0)
        def _():
            stage = soa_v.at[0, pl.ds(0, n_pre)]
            for j in range(3):
                pltpu.sync_copy(p_in_t.at[j], stage)
                pltpu.sync_copy(stage, p_out.at[j, pl.ds(0, n_pre)])
                pltpu.sync_copy(v_in_t.at[j], stage)
                pltpu.sync_copy(stage, v_out.at[j, pl.ds(0, n_pre)])

        lane = lax.iota(jnp.int32, _L)

        def chunk_body(g, carry):
            base = wid * R + g * C
            pltpu.sync_copy(idx.at[pl.ds(base, C)], idx_v)
            pltpu.async_copy(table.at[idx_v], rows_v, sem).wait()

            def grp(i, carry2):
                row = lane + i * _L
                sl = pl.ds(i * _L, _L)
                for jj in range(5):
                    soa_v[jj, sl] = plsc.load_gather(
                        rows_v, [row, jnp.full((_L,), jj + 1, jnp.int32)])
                return carry2

            lax.fori_loop(0, C // _L, grp, 0)
            dst = pl.ds(n_pre + base, C)
            pltpu.sync_copy(zero_v, p_out.at[0, dst])
            pltpu.sync_copy(soa_v.at[0], p_out.at[1, dst])
            pltpu.sync_copy(soa_v.at[1], p_out.at[2, dst])
            pltpu.sync_copy(soa_v.at[2], v_out.at[0, dst])
            pltpu.sync_copy(soa_v.at[3], v_out.at[1, dst])
            pltpu.sync_copy(soa_v.at[4], v_out.at[2, dst])
            return carry

        lax.fori_loop(0, n_chunks, chunk_body, 0)

    p_soa, v_soa = sample(t8, indices.astype(jnp.int32), P_in.T, V_in.T)
    return (p_soa.T, v_soa.T)


# speedup vs baseline: 6.8932x; 2.6757x over previous
import functools

import jax
import jax.numpy as jnp
from jax import lax
from jax.experimental import pallas as pl
from jax.experimental.pallas import tpu as pltpu
from jax.experimental.pallas import tpu_sc as plsc

_L = 16


def _tc_transform(t_t, blk):
    n_tab = t_t.shape[1]
    grid = (n_tab + blk - 1) // blk

    def body(in_ref, p1_ref, p2_ref, v0_ref, v1_ref, v2_ref):
        r = in_ref[...]
        r3 = r[3, :]
        r4 = r[4, :]
        r5 = r[5, :]
        s = r3 * r3 + r4 * r4 + r5 * r5
        norm = jnp.sqrt(s)
        inv = 1.0 / jnp.maximum(norm, jnp.float32(1e-12))
        p1_ref[...] = 1000.0 * r[0, :]
        p2_ref[...] = 1000.0 * r[1, :]
        v0_ref[...] = -r5 * inv
        v1_ref[...] = r3 * inv
        v2_ref[...] = r4 * inv

    out_sds = jax.ShapeDtypeStruct((n_tab,), jnp.float32)
    return pl.pallas_call(
        body,
        grid=(grid,),
        in_specs=[pl.BlockSpec((6, blk), lambda i: (0, i))],
        out_specs=[pl.BlockSpec((blk,), lambda i: (i,))] * 5,
        out_shape=[out_sds] * 5,
    )(t_t)


def kernel(all_rays, indices, P_in, V_in):
    n_tab = all_rays.shape[0]
    B = indices.shape[0]
    n_pre = P_in.shape[0]
    info = plsc.get_sparse_core_info()
    NC, NS = info.num_cores, info.num_subcores
    NW = NC * NS
    mesh_kw = dict(core_axis_name="c", subcore_axis_name="s")

    comps = _tc_transform(all_rays.T, 32768)

    KC = 8000
    n_ck = n_tab // KC

    @functools.partial(
        pl.kernel,
        out_type=jax.ShapeDtypeStruct((n_tab * 8,), jnp.float32),
        mesh=plsc.VectorSubcoreMesh(**mesh_kw),
        scratch_types=[
            pltpu.VMEM((5, KC), jnp.float32),
            pltpu.VMEM((KC * 8,), jnp.float32),
        ],
        compiler_params=pltpu.CompilerParams(
            needs_layout_passes=False, use_tc_tiling_on_sc=False),
    )
    def interleave(p1, p2, v0, v1, v2, t8, in_v, out_v):
        wid = lax.axis_index("s") * NC + lax.axis_index("c")
        lane8 = lax.iota(jnp.int32, _L) * 8

        def chunk_body(ck, carry):
            c = wid + ck * NW

            @pl.when(c < n_ck)
            def _():
                start = c * KC
                for j, comp in enumerate((p1, p2, v0, v1, v2)):
                    pltpu.sync_copy(comp.at[pl.ds(start, KC)], in_v.at[j])

                def grp(i, carry2):
                    sl = pl.ds(i * _L, _L)
                    dst = lane8 + i * (_L * 8)
                    for j in range(5):
                        plsc.store_scatter(out_v, [dst + j + 1], in_v[j, sl])
                    return carry2

                lax.fori_loop(0, KC // _L, grp, 0)
                pltpu.sync_copy(out_v, t8.at[pl.ds(start * 8, KC * 8)])

            return carry

        lax.fori_loop(0, (n_ck + NW - 1) // NW, chunk_body, 0)

    t8 = interleave(*comps).reshape(n_tab, 8)

    R = B // NW
    C = 4096
    n_chunks = R // C
    n_out = B + n_pre
    out_sds = jax.ShapeDtypeStruct((3, n_out), jnp.float32)

    @functools.partial(
        pl.kernel,
        out_type=(out_sds, out_sds),
        mesh=plsc.VectorSubcoreMesh(**mesh_kw),
        scratch_types=[
            pltpu.VMEM((C,), jnp.int32),
            pltpu.VMEM((C, 8), jnp.float32),
            pltpu.VMEM((5, C), jnp.float32),
            pltpu.VMEM((C,), jnp.float32),
            pltpu.SemaphoreType.DMA,
        ],
        compiler_params=pltpu.CompilerParams(
            needs_layout_passes=False, use_tc_tiling_on_sc=False),
    )
    def sample(table, idx, p_in_t, v_in_t, p_out, v_out,
               idx_v, rows_v, soa_v, zero_v, sem):
        wid = lax.axis_index("s") * NC + lax.axis_index("c")
        zvec = jnp.zeros((_L,), jnp.float32)

        def zinit(i, carry):
            zero_v[pl.ds(i * _L, _L)] = zvec
            return carry

        lax.fori_loop(0, C // _L, zinit, 0)

        @pl.when(wid == 0)
        def _():
            stage = soa_v.at[0, pl.ds(0, n_pre)]
            for j in range(3):
                pltpu.sync_copy(p_in_t.at[j], stage)
                pltpu.sync_copy(stage, p_out.at[j, pl.ds(0, n_pre)])
                pltpu.sync_copy(v_in_t.at[j], stage)
                pltpu.sync_copy(stage, v_out.at[j, pl.ds(0, n_pre)])

        lane = lax.iota(jnp.int32, _L)

        def chunk_body(g, carry):
            base = wid * R + g * C
            pltpu.sync_copy(idx.at[pl.ds(base, C)], idx_v)
            pltpu.async_copy(table.at[idx_v], rows_v, sem).wait()

            def grp(i, carry2):
                row = lane + i * _L
                sl = pl.ds(i * _L, _L)
                for jj in range(5):
                    soa_v[jj, sl] = plsc.load_gather(
                        rows_v, [row, jnp.full((_L,), jj + 1, jnp.int32)])
                return carry2

            lax.fori_loop(0, C // _L, grp, 0)
            dst = pl.ds(n_pre + base, C)
            pltpu.sync_copy(zero_v, p_out.at[0, dst])
            pltpu.sync_copy(soa_v.at[0], p_out.at[1, dst])
            pltpu.sync_copy(soa_v.at[1], p_out.at[2, dst])
            pltpu.sync_copy(soa_v.at[2], v_out.at[0, dst])
            pltpu.sync_copy(soa_v.at[3], v_out.at[1, dst])
            pltpu.sync_copy(soa_v.at[4], v_out.at[2, dst])
            return carry

        lax.fori_loop(0, n_chunks, chunk_body, 0)

    p_soa, v_soa = sample(t8, indices.astype(jnp.int32), P_in.T, V_in.T)
    return (p_soa.T, v_soa.T)
